# baseline (speedup 1.0000x reference)
import jax
import jax.numpy as jnp
from jax.experimental import pallas as pl
from jax.experimental.pallas import tpu as pltpu

_B = 4096
_T = 750
_WIN = 11
_SIGMA = 1.0
_E_THETA = 0.1
_E_G = 1.0
_E_ALPHA = 1.0
_TC = 128
_G = 6
_PW = (_G + 1) * _TC
_CB = 128
_AEH = 5376
_NS_THRESH = 350.0


def _loss_body(p4a_ref, p4b_ref, p3a_ref, p3b_ref, out_ref, g_ref, ae_ref):
    g = pl.program_id(0)

    def win(aref, bref, r, j):
        rows = pl.ds(r, _CB)
        if j == 0:
            return aref[rows, :]
        return jnp.concatenate([aref[rows, j:], bref[rows, :j]], axis=1)

    def fold8(x):
        return jnp.sum(x.reshape(_CB // 8, 8, _TC), axis=0)

    lane = jax.lax.broadcasted_iota(jnp.int32, (1, _TC), 1) + g * _TC
    valid = lane < _T
    ns_bias = jnp.where(valid, 0.0, jnp.float32(1e9))

    inv_two_sigma2 = jnp.float32(-0.5 / (_SIGMA * _SIGMA))
    dn = (((0,), (0,)), ((), ()))

    a4 = p4a_ref[...]
    b4 = p4b_ref[...]
    g_aa = jax.lax.dot_general(a4, a4, dn, preferred_element_type=jnp.float32)
    g_ab = jax.lax.dot_general(a4, b4, dn, preferred_element_type=jnp.float32)
    g_bb = jax.lax.dot_general(b4, b4, dn, preferred_element_type=jnp.float32)
    g_ref[:_TC, :_TC] = g_aa
    g_ref[:_TC, _TC:] = g_ab
    g_ref[_TC:, :_TC] = g_ab.T
    g_ref[_TC:, _TC:] = g_bb

    rr = jax.lax.broadcasted_iota(jnp.int32, (_TC, _TC), 0)
    cc = jax.lax.broadcasted_iota(jnp.int32, (_TC, _TC), 1)
    eye = (rr == cc).astype(jnp.float32)

    def diag_at(row0, col0):
        blk = g_ref[row0:row0 + _TC, col0:col0 + _TC]
        return jnp.sum(blk * eye, axis=0, keepdims=True)

    cs_a = diag_at(0, 0)
    cs_b = diag_at(_TC, _TC)
    cs = jnp.concatenate([cs_a, cs_b], axis=1)
    cs6 = cs[:, 6:6 + _TC]
    min_ns = None
    for j in range(_WIN):
        if j == 6:
            continue
        nsj = cs6 + cs[:, j:j + _TC] - 2.0 * diag_at(6, j) + ns_bias
        min_ns = nsj if min_ns is None else jnp.minimum(min_ns, nsj)
    any_live = jnp.min(min_ns) < jnp.float32(_NS_THRESH)

    pcol = lane
    tvalid = (pcol >= 6) & (pcol < _T + 6)

    def th_chunk(i, acc):
        r = pl.ds(i * _CB, _CB)
        d = p4a_ref[r, :] - p3a_ref[r, :]
        return acc + fold8(d * d)

    th = jax.lax.fori_loop(0, _B // _CB, th_chunk,
                           jnp.zeros((8, _TC), jnp.float32))
    base = jnp.sum(th, axis=0, keepdims=True) * jnp.float32(_E_THETA)
    out_ref[...] = jnp.where(tvalid, base, 0.0).reshape(1, 1, _TC)

    @pl.when(any_live)
    def _():
        jlist = [j for j in range(_WIN) if j != 6]

        def ns_chunk(i, carry):
            r = i * _CB
            ac = win(p4a_ref, p4b_ref, r, 6)
            new = [None] * len(jlist)
            for jj, j in enumerate(jlist):
                d = ac - win(p4a_ref, p4b_ref, r, j)
                new[jj] = carry[jj] + fold8(d * d)
            return tuple(new)

        zeros = jnp.zeros((8, _TC), jnp.float32)
        ns_acc = jax.lax.fori_loop(0, _B // _CB, ns_chunk,
                                   (zeros,) * len(jlist))
        ens = [jnp.exp(inv_two_sigma2 *
                       (jnp.sum(ns_acc[jj], axis=0, keepdims=True) + ns_bias))
               for jj in range(len(jlist))]

        def ae_fill(i, _):
            src = jax.lax.rem(i * _CB, jnp.int32(_B))
            ae_ref[pl.ds(i * _CB, _CB), :] = win(p4a_ref, p4b_ref, src, 6)
            return 0

        jax.lax.fori_loop(0, _AEH // _CB, ae_fill, 0)

        def l1_chunk(i, tot):
            r = i * _CB
            s = jax.lax.rem(jnp.int32(11) * _CB * i, jnp.int32(_B))
            mw = (ae_ref[pl.Slice(s, _CB, _WIN), :]
                  - win(p4a_ref, p4b_ref, r, 0))
            for j in range(1, _WIN):
                mw = jnp.maximum(
                    mw, ae_ref[pl.Slice(s + j, _CB, _WIN), :]
                    - win(p4a_ref, p4b_ref, r, j))
            eg = jnp.exp(inv_two_sigma2 * jnp.float32(_E_G) * mw * mw)
            a2 = win(p3a_ref, p3b_ref, r, 6)
            acc = None
            for jj, j in enumerate(jlist):
                t = jnp.minimum(ens[jj], eg) * jnp.abs(
                    a2 - win(p3a_ref, p3b_ref, r, j))
                acc = t if acc is None else acc + t
            return tot + fold8(acc)

        tot = jax.lax.fori_loop(0, _B // _CB, l1_chunk,
                                jnp.zeros((8, _TC), jnp.float32))
        part = jnp.sum(tot, axis=0, keepdims=True)
        out_ref[...] += jnp.where(valid, part, 0.0).reshape(1, 1, _TC)


def _pad_like_ref(x):
    b = x.shape[0]
    front = jnp.tile(x[:, 0], 6).reshape(b, 6)
    back = jnp.tile(x[:, -1], 6).reshape(b, 6)
    p = jax.lax.pad(x, jnp.float32(0.0), ((0, 0, 0), (6, _PW - _T - 6, 0)))
    p = jax.lax.dynamic_update_slice(p, front, (0, 0))
    p = jax.lax.dynamic_update_slice(p, back[:, 1:], (0, 6 + _T))
    return p


def kernel(actioness, actioness_2):
    p4 = _pad_like_ref(actioness)
    p3 = _pad_like_ref(actioness_2)

    col = pl.BlockSpec((_B, _TC), lambda i: (0, i))
    col_next = pl.BlockSpec((_B, _TC), lambda i: (0, i + 1))

    partials = pl.pallas_call(
        _loss_body,
        grid=(_G,),
        in_specs=[col, col_next, col, col_next],
        out_specs=pl.BlockSpec((1, 1, _TC), lambda i: (i, 0, 0)),
        out_shape=jax.ShapeDtypeStruct((_G, 1, _TC), jnp.float32),
        scratch_shapes=[
            pltpu.VMEM((2 * _TC, 2 * _TC), jnp.float32),
            pltpu.VMEM((_AEH, _TC), jnp.float32),
        ],
        compiler_params=pltpu.CompilerParams(
            dimension_semantics=("parallel",),
            vmem_limit_bytes=48 * 1024 * 1024,
        ),
        name="act_eloss_v3",
    )(p4, p4, p3, p3)

    return jnp.float32(_E_ALPHA / _B) * jnp.sum(partials)


# speedup vs baseline: 1.5860x; 1.5860x over previous
import jax
import jax.numpy as jnp
from jax.experimental import pallas as pl
from jax.experimental.pallas import tpu as pltpu

_B = 4096
_T = 750
_WIN = 11
_SIGMA = 1.0
_E_THETA = 0.1
_E_G = 1.0
_E_ALPHA = 1.0
_TC = 128
_G = 6
_PW = (_G + 1) * _TC
_CB = 128
_AEH = 5376
_NS_THRESH = 350.0


def _loss_body(p4a_ref, p4b_ref, p3a_ref, p3b_ref, out_ref, g_ref, ae_ref):
    g = pl.program_id(0)

    def win(aref, bref, r, j):
        rows = pl.ds(r, _CB)
        if j == 0:
            return aref[rows, :]
        return jnp.concatenate([aref[rows, j:], bref[rows, :j]], axis=1)

    def fold8(x):
        return jnp.sum(x.reshape(_CB // 8, 8, _TC), axis=0)

    lane = jax.lax.broadcasted_iota(jnp.int32, (1, _TC), 1) + g * _TC
    valid = lane < _T
    ns_bias = jnp.where(valid, 0.0, jnp.float32(1e9))

    inv_two_sigma2 = jnp.float32(-0.5 / (_SIGMA * _SIGMA))
    dn = (((0,), (0,)), ((), ()))

    a4 = p4a_ref[...]
    b4 = p4b_ref[...]
    g_aa = jax.lax.dot_general(a4, a4, dn, preferred_element_type=jnp.float32)
    g_ab = jax.lax.dot_general(a4, b4, dn, preferred_element_type=jnp.float32)
    g_bb = jax.lax.dot_general(b4, b4, dn, preferred_element_type=jnp.float32)
    g_ref[:_TC, :_TC] = g_aa
    g_ref[:_TC, _TC:] = g_ab
    g_ref[_TC:, :_TC] = g_ab.T
    g_ref[_TC:, _TC:] = g_bb

    rr = jax.lax.broadcasted_iota(jnp.int32, (_TC, _TC), 0)
    cc = jax.lax.broadcasted_iota(jnp.int32, (_TC, _TC), 1)
    eye = (rr == cc).astype(jnp.float32)

    def diag_at(row0, col0):
        blk = g_ref[row0:row0 + _TC, col0:col0 + _TC]
        return jnp.sum(blk * eye, axis=0, keepdims=True)

    cs_a = diag_at(0, 0)
    cs_b = diag_at(_TC, _TC)
    cs = jnp.concatenate([cs_a, cs_b], axis=1)
    cs6 = cs[:, 6:6 + _TC]
    min_ns = None
    for j in range(_WIN):
        if j == 6:
            continue
        nsj = cs6 + cs[:, j:j + _TC] - 2.0 * diag_at(6, j) + ns_bias
        min_ns = nsj if min_ns is None else jnp.minimum(min_ns, nsj)
    any_live = jnp.min(min_ns) < jnp.float32(_NS_THRESH)

    pcol = lane
    tvalid = (pcol >= 6) & (pcol < _T + 6)

    def th_chunk(i, acc):
        r = pl.ds(i * _CB, _CB)
        d = p4a_ref[r, :] - p3a_ref[r, :]
        return acc + fold8(d * d)

    th = jax.lax.fori_loop(0, _B // _CB, th_chunk,
                           jnp.zeros((8, _TC), jnp.float32))
    base = jnp.sum(th, axis=0, keepdims=True) * jnp.float32(_E_THETA)
    out_ref[...] = jnp.where(tvalid, base, 0.0).reshape(1, 1, _TC)

    @pl.when(any_live)
    def _():
        jlist = [j for j in range(_WIN) if j != 6]

        def ns_chunk(i, carry):
            r = i * _CB
            ac = win(p4a_ref, p4b_ref, r, 6)
            new = [None] * len(jlist)
            for jj, j in enumerate(jlist):
                d = ac - win(p4a_ref, p4b_ref, r, j)
                new[jj] = carry[jj] + fold8(d * d)
            return tuple(new)

        zeros = jnp.zeros((8, _TC), jnp.float32)
        ns_acc = jax.lax.fori_loop(0, _B // _CB, ns_chunk,
                                   (zeros,) * len(jlist))
        ens = [jnp.exp(inv_two_sigma2 *
                       (jnp.sum(ns_acc[jj], axis=0, keepdims=True) + ns_bias))
               for jj in range(len(jlist))]

        def ae_fill(i, _):
            src = jax.lax.rem(i * _CB, jnp.int32(_B))
            ae_ref[pl.ds(i * _CB, _CB), :] = win(p4a_ref, p4b_ref, src, 6)
            return 0

        jax.lax.fori_loop(0, _AEH // _CB, ae_fill, 0)

        def l1_chunk(i, tot):
            r = i * _CB
            s = jax.lax.rem(jnp.int32(11) * _CB * i, jnp.int32(_B))
            mw = (ae_ref[pl.Slice(s, _CB, _WIN), :]
                  - win(p4a_ref, p4b_ref, r, 0))
            for j in range(1, _WIN):
                mw = jnp.maximum(
                    mw, ae_ref[pl.Slice(s + j, _CB, _WIN), :]
                    - win(p4a_ref, p4b_ref, r, j))
            eg = jnp.exp(inv_two_sigma2 * jnp.float32(_E_G) * mw * mw)
            a2 = win(p3a_ref, p3b_ref, r, 6)
            acc = None
            for jj, j in enumerate(jlist):
                t = jnp.minimum(ens[jj], eg) * jnp.abs(
                    a2 - win(p3a_ref, p3b_ref, r, j))
                acc = t if acc is None else acc + t
            return tot + fold8(acc)

        tot = jax.lax.fori_loop(0, _B // _CB, l1_chunk,
                                jnp.zeros((8, _TC), jnp.float32))
        part = jnp.sum(tot, axis=0, keepdims=True)
        out_ref[...] += jnp.where(valid, part, 0.0).reshape(1, 1, _TC)


def _pad_like_ref(x):
    b = x.shape[0]
    front = jnp.tile(x[:, 0], 6).reshape(b, 6)
    back = jnp.tile(x[:, -1], 6).reshape(b, 6)
    zpad = jnp.zeros((b, _PW - (_T + _WIN)), x.dtype)
    return jnp.concatenate([front, x, back[:, 1:], zpad], axis=1)


def kernel(actioness, actioness_2):
    p4 = _pad_like_ref(actioness)
    p3 = _pad_like_ref(actioness_2)

    col = pl.BlockSpec((_B, _TC), lambda i: (0, i))
    col_next = pl.BlockSpec((_B, _TC), lambda i: (0, i + 1))

    partials = pl.pallas_call(
        _loss_body,
        grid=(_G,),
        in_specs=[col, col_next, col, col_next],
        out_specs=pl.BlockSpec((1, 1, _TC), lambda i: (i, 0, 0)),
        out_shape=jax.ShapeDtypeStruct((_G, 1, _TC), jnp.float32),
        scratch_shapes=[
            pltpu.VMEM((2 * _TC, 2 * _TC), jnp.float32),
            pltpu.VMEM((_AEH, _TC), jnp.float32),
        ],
        compiler_params=pltpu.CompilerParams(
            dimension_semantics=("parallel",),
            vmem_limit_bytes=48 * 1024 * 1024,
        ),
        name="act_eloss_v3",
    )(p4, p4, p3, p3)

    return jnp.float32(_E_ALPHA / _B) * jnp.sum(partials)
